# baseline (speedup 1.0000x reference)
import functools

import jax
import jax.numpy as jnp
from jax.experimental import pallas as pl

_EPS = 1e-6
_CAP_FACTOR = 1.25


def _gate_kernel(x_ref, w_ref, b_ref, o_ref, *, capacity):
    batch, sblk, dim = x_ref.shape
    e = w_ref.shape[0]
    x = x_ref[...].reshape(batch * sblk, dim)
    logits = jax.lax.dot_general(
        x, w_ref[...], (((1,), (1,)), ((), ())),
        preferred_element_type=jnp.float32)
    logits = logits + b_ref[...]

    m = jnp.max(logits, axis=-1, keepdims=True)
    ex = jnp.exp(logits - m)
    probs = ex / jnp.sum(ex, axis=-1, keepdims=True)

    iota = jax.lax.broadcasted_iota(jnp.int32, logits.shape, 1)
    i1 = jnp.min(jnp.where(logits == m, iota, e), axis=-1, keepdims=True)
    mask1 = iota == i1
    neg = jnp.float32(-jnp.inf)
    l2 = jnp.where(mask1, neg, logits)
    m2 = jnp.max(l2, axis=-1, keepdims=True)
    i2 = jnp.min(jnp.where(l2 == m2, iota, e), axis=-1, keepdims=True)
    mask = mask1 | (iota == i2)

    masked = jnp.where(mask, probs, jnp.float32(0.0))
    md = masked.reshape(batch, sblk, e)
    den = jnp.sum(md, axis=0, keepdims=True) + jnp.float32(_EPS)
    o_ref[...] = md / den * jnp.float32(capacity)


def kernel(X, Wg, bg):
    batch, seq, dim = X.shape
    e = Wg.shape[0]
    capacity = int(_CAP_FACTOR * batch)
    sblk = 128
    grid = (seq // sblk,)
    out = pl.pallas_call(
        functools.partial(_gate_kernel, capacity=capacity),
        grid=grid,
        in_specs=[
            pl.BlockSpec((batch, sblk, dim), lambda i: (0, i, 0)),
            pl.BlockSpec((e, dim), lambda i: (0, 0)),
            pl.BlockSpec((1, e), lambda i: (0, 0)),
        ],
        out_specs=pl.BlockSpec((batch, sblk, e), lambda i: (0, i, 0)),
        out_shape=jax.ShapeDtypeStruct((batch, seq, e), jnp.float32),
    )(X, Wg, bg.reshape(1, e))
    return (out, None)


# speedup vs baseline: 1.1477x; 1.1477x over previous
import functools

import jax
import jax.numpy as jnp
from jax.experimental import pallas as pl

_EPS = 1e-6
_CAP_FACTOR = 1.25


def _gate_kernel(x_ref, w_ref, b_ref, o_ref, *, capacity):
    batch, sblk, dim = x_ref.shape
    e = w_ref.shape[0]
    x = x_ref[...].reshape(batch * sblk, dim)
    logits = jax.lax.dot_general(
        w_ref[...], x, (((1,), (1,)), ((), ())),
        preferred_element_type=jnp.float32)
    logits = logits + jnp.transpose(b_ref[...], (1, 0))

    m = jnp.max(logits, axis=0, keepdims=True)
    ex = jnp.exp(logits - m)
    probs = ex / jnp.sum(ex, axis=0, keepdims=True)

    iota = jax.lax.broadcasted_iota(jnp.int32, logits.shape, 0)
    i1 = jnp.min(jnp.where(logits == m, iota, e), axis=0, keepdims=True)
    mask1 = iota == i1
    neg = jnp.float32(-jnp.inf)
    l2 = jnp.where(mask1, neg, logits)
    m2 = jnp.max(l2, axis=0, keepdims=True)
    i2 = jnp.min(jnp.where(l2 == m2, iota, e), axis=0, keepdims=True)
    mask = mask1 | (iota == i2)

    masked = jnp.where(mask, probs, jnp.float32(0.0))
    den = jnp.float32(_EPS)
    for b in range(batch):
        den = den + masked[:, b * sblk:(b + 1) * sblk]
    scale = jnp.float32(capacity) / den
    for b in range(batch):
        o_ref[b] = masked[:, b * sblk:(b + 1) * sblk] * scale


def kernel(X, Wg, bg):
    batch, seq, dim = X.shape
    e = Wg.shape[0]
    capacity = int(_CAP_FACTOR * batch)
    sblk = 256
    grid = (seq // sblk,)
    out = pl.pallas_call(
        functools.partial(_gate_kernel, capacity=capacity),
        grid=grid,
        in_specs=[
            pl.BlockSpec((batch, sblk, dim), lambda i: (0, i, 0)),
            pl.BlockSpec((e, dim), lambda i: (0, 0)),
            pl.BlockSpec((1, e), lambda i: (0, 0)),
        ],
        out_specs=pl.BlockSpec((batch, e, sblk), lambda i: (0, 0, i)),
        out_shape=jax.ShapeDtypeStruct((batch, e, seq), jnp.float32),
    )(X, Wg, bg.reshape(1, e))
    return (jnp.transpose(out, (0, 2, 1)), None)
